# baseline (speedup 1.0000x reference)
import functools

import jax
import jax.numpy as jnp
from jax import lax
from jax.experimental import pallas as pl
from jax.experimental.pallas import tpu as pltpu
from jax.experimental.pallas import tpu_sc as plsc

NC = 2
NS = 16
NW = NC * NS
LANES = 16
WAVES = (8, 16, 16, 16, 8)
ROWS_MAX = max(WAVES)


def _emb_body(batch, seq, s_per_w, d,
              ids_hbm, tok_hbm, pos_hbm, out_hbm,
              idx_v, tok_v, pos_v, *sems):
    n_waves = len(WAVES)
    offs = [sum(WAVES[:i]) for i in range(n_waves)]
    gsems = (sems[:batch], sems[batch:2 * batch])
    ssems = (sems[2 * batch:3 * batch], sems[3 * batch:4 * batch])
    psems = sems[4 * batch:4 * batch + 2]
    wid = lax.axis_index("s") * NC + lax.axis_index("c")
    seq0 = wid * s_per_w

    isem = psems[1]
    idx_fetches = [pltpu.async_copy(
        ids_hbm.at[pl.ds(b * seq + seq0, s_per_w)], idx_v.at[b], isem)
        for b in range(batch)]
    for f in idx_fetches:
        f.wait()

    def start_wave(w):
        wb = w % 2
        h0, rw = offs[w], WAVES[w]
        fetches = [pltpu.async_copy(
            pos_hbm.at[pl.ds(seq0 + h0, rw)],
            pos_v.at[wb, pl.ds(0, rw)], psems[wb])]
        for b in range(batch):
            fetches.append(pltpu.async_copy(
                tok_hbm.at[idx_v.at[b, pl.ds(h0, rw)]],
                tok_v.at[wb, b, pl.ds(0, rw)], gsems[wb][b]))
        return fetches

    fetches = {0: start_wave(0)}
    stores = {}
    for w in range(n_waves):
        wb = w % 2
        h0, rw = offs[w], WAVES[w]
        if w >= 1:
            for st in stores.pop(w - 1):
                st.wait()
        if w + 1 < n_waves:
            fetches[w + 1] = start_wave(w + 1)
        for f in fetches.pop(w):
            f.wait()

        stores[w] = []
        for half in range(2):
            bs = (half * batch // 2, half * batch // 2 + batch // 2)

            @plsc.parallel_loop(0, rw, unroll=1)
            def p_body(p):
                for c in range(d // LANES):
                    sl = pl.ds(c * LANES, LANES)
                    x = pos_v[wb, p, sl]
                    for b in range(*bs):
                        plsc.addupdate(tok_v.at[wb, b, p, sl], x)

            stores[w] += [pltpu.async_copy(
                tok_v.at[wb, b, pl.ds(0, rw)],
                out_hbm.at[pl.ds(b * seq + seq0 + h0, rw)], ssems[wb][b])
                for b in range(*bs)]
    for w in sorted(stores):
        for st in stores[w]:
            st.wait()


def kernel(input_ids, token_table, pos_table):
    batch, seq = input_ids.shape
    vocab, d = token_table.shape
    n = batch * seq
    ids_flat = input_ids.reshape(n).astype(jnp.int32)

    s_per_w = seq // NW
    assert sum(WAVES) == s_per_w

    mesh = plsc.VectorSubcoreMesh(core_axis_name="c", subcore_axis_name="s")

    run = functools.partial(
        pl.kernel,
        mesh=mesh,
        out_type=jax.ShapeDtypeStruct((n, d), jnp.float32),
        scratch_types=[
            pltpu.VMEM((batch, s_per_w), jnp.int32),
            pltpu.VMEM((2, batch, ROWS_MAX, d), jnp.float32),
            pltpu.VMEM((2, ROWS_MAX, d), jnp.float32),
        ] + [pltpu.SemaphoreType.DMA] * (4 * batch + 2),
    )(functools.partial(_emb_body, batch, seq, s_per_w, d))

    out = run(ids_flat, token_table, pos_table)
    return out.reshape(batch, seq, d)


# speedup vs baseline: 1.0312x; 1.0312x over previous
import functools

import jax
import jax.numpy as jnp
from jax import lax
from jax.experimental import pallas as pl
from jax.experimental.pallas import tpu as pltpu
from jax.experimental.pallas import tpu_sc as plsc

NC = 2
NS = 16
NW = NC * NS
LANES = 16
WAVES = (8, 16, 16, 16, 8)
ROWS_MAX = max(WAVES)


def _emb_body(batch, seq, s_per_w, d,
              ids_hbm, tok_hbm, pos_hbm, out_hbm,
              idx_v, tok_v, pos_v, *sems):
    n_waves = len(WAVES)
    offs = [sum(WAVES[:i]) for i in range(n_waves)]
    gsems = (sems[:batch], sems[batch:2 * batch])
    ssems = (sems[2 * batch:3 * batch], sems[3 * batch:4 * batch])
    psems = sems[4 * batch:4 * batch + 2]
    wid = lax.axis_index("s") * NC + lax.axis_index("c")
    seq0 = wid * s_per_w

    isem = psems[1]
    idx_fetches = [pltpu.async_copy(
        ids_hbm.at[pl.ds(b * seq + seq0, s_per_w)], idx_v.at[b], isem)
        for b in range(batch)]
    for f in idx_fetches:
        f.wait()

    def start_pos(w):
        wb = w % 2
        h0, rw = offs[w], WAVES[w]
        return pltpu.async_copy(
            pos_hbm.at[pl.ds(seq0 + h0, rw)],
            pos_v.at[wb, pl.ds(0, rw)], psems[wb])

    def start_gathers(w):
        wb = w % 2
        h0, rw = offs[w], WAVES[w]
        return [pltpu.async_copy(
            tok_hbm.at[idx_v.at[b, pl.ds(h0, rw)]],
            tok_v.at[wb, b, pl.ds(0, rw)], gsems[wb][b])
            for b in range(batch)]

    fetches = {0: start_gathers(0) + [start_pos(0)]}
    stores = {}
    for w in range(n_waves):
        wb = w % 2
        h0, rw = offs[w], WAVES[w]
        if w + 1 < n_waves:
            pfetch = [start_pos(w + 1)]
        if w >= 1:
            for st in stores.pop(w - 1):
                st.wait()
        if w + 1 < n_waves:
            fetches[w + 1] = start_gathers(w + 1) + pfetch
        for f in fetches.pop(w):
            f.wait()

        @plsc.parallel_loop(0, rw, unroll=1)
        def p_body(p):
            for c in range(d // LANES):
                sl = pl.ds(c * LANES, LANES)
                x = pos_v[wb, p, sl]
                for b in range(batch):
                    plsc.addupdate(tok_v.at[wb, b, p, sl], x)

        stores[w] = [pltpu.async_copy(
            tok_v.at[wb, b, pl.ds(0, rw)],
            out_hbm.at[pl.ds(b * seq + seq0 + h0, rw)], ssems[wb][b])
            for b in range(batch)]
    for w in sorted(stores):
        for st in stores[w]:
            st.wait()


def kernel(input_ids, token_table, pos_table):
    batch, seq = input_ids.shape
    vocab, d = token_table.shape
    n = batch * seq
    ids_flat = input_ids.reshape(n).astype(jnp.int32)

    s_per_w = seq // NW
    assert sum(WAVES) == s_per_w

    mesh = plsc.VectorSubcoreMesh(core_axis_name="c", subcore_axis_name="s")

    run = functools.partial(
        pl.kernel,
        mesh=mesh,
        out_type=jax.ShapeDtypeStruct((n, d), jnp.float32),
        scratch_types=[
            pltpu.VMEM((batch, s_per_w), jnp.int32),
            pltpu.VMEM((2, batch, ROWS_MAX, d), jnp.float32),
            pltpu.VMEM((2, ROWS_MAX, d), jnp.float32),
        ] + [pltpu.SemaphoreType.DMA] * (4 * batch + 2),
    )(functools.partial(_emb_body, batch, seq, s_per_w, d))

    out = run(ids_flat, token_table, pos_table)
    return out.reshape(batch, seq, d)
